# baseline (speedup 1.0000x reference)
import functools

import jax
import jax.numpy as jnp
from jax import lax
from jax.experimental import pallas as pl
from jax.experimental.pallas import tpu as pltpu
from jax.experimental.pallas import tpu_sc as plsc

B = 16384
D = 128
L = 16
NC = 2
NS = 16
NW = NC * NS
ROWS_PER_W = B // NW
CHUNK = 128
NCHUNK = ROWS_PER_W // CHUNK


def _rsqrt_newton(x):
    i = lax.bitcast_convert_type(x, jnp.int32)
    i = jnp.int32(0x5F3759DF) - lax.shift_right_logical(i, 1)
    y = lax.bitcast_convert_type(i, jnp.float32)
    half_x = jnp.float32(0.5) * x
    for _ in range(3):
        y = y * (jnp.float32(1.5) - half_x * y * y)
    return y


def _make_sc_kernel():
    mesh = plsc.VectorSubcoreMesh(core_axis_name="c", subcore_axis_name="s")

    @functools.partial(
        pl.kernel,
        mesh=mesh,
        compiler_params=pltpu.CompilerParams(needs_layout_passes=False),
        out_type=jax.ShapeDtypeStruct((B,), jnp.float32),
        scratch_types=[
            pltpu.VMEM((NCHUNK, CHUNK), jnp.int32),
            pltpu.VMEM((CHUNK, D), jnp.float32),
            pltpu.VMEM((CHUNK, D), jnp.float32),
            pltpu.VMEM((CHUNK, D), jnp.float32),
            pltpu.VMEM((CHUNK,), jnp.float32),
            pltpu.SemaphoreType.DMA,
        ],
    )
    def sc_kernel(src_hbm, ids_hbm, dst_hbm, table_hbm, out_hbm,
                  idx_v, src_v, dst_v, rel_v, out_v, sem):
        wid = lax.axis_index("s") * NC + lax.axis_index("c")
        base = wid * ROWS_PER_W
        for c in range(NCHUNK):
            r0 = base + c * CHUNK
            pltpu.sync_copy(ids_hbm.at[pl.ds(r0, CHUNK)], idx_v.at[c])
            pltpu.async_copy(table_hbm.at[idx_v.at[c]], rel_v, sem).wait()
            pltpu.sync_copy(src_hbm.at[pl.ds(r0, CHUNK)], src_v)
            pltpu.sync_copy(dst_hbm.at[pl.ds(r0, CHUNK)], dst_v)
            for g in range(CHUNK // L):
                rows = jnp.arange(L, dtype=jnp.int32) + jnp.int32(g * L)

                def body(j, acc):
                    cols = jnp.full((L,), j, dtype=jnp.int32)
                    s = plsc.load_gather(src_v, [rows, cols])
                    r = plsc.load_gather(rel_v, [rows, cols])
                    d = plsc.load_gather(dst_v, [rows, cols])
                    t = s + r - d
                    return acc + t * t

                acc = lax.fori_loop(0, D, body, jnp.zeros((L,), jnp.float32))
                x = jnp.maximum(acc, jnp.float32(1e-30))
                out_v[pl.ds(g * L, L)] = -(x * _rsqrt_newton(x))
            pltpu.sync_copy(out_v, out_hbm.at[pl.ds(r0, CHUNK)])

    return sc_kernel


_SC_KERNEL = _make_sc_kernel()


def kernel(src_emb, rel_ids, dst_emb, rel_table):
    ids = rel_ids.astype(jnp.int32)
    return _SC_KERNEL(src_emb, ids, dst_emb, rel_table)


# speedup vs baseline: 1.1871x; 1.1871x over previous
import functools

import jax
import jax.numpy as jnp
from jax import lax
from jax.experimental import pallas as pl
from jax.experimental.pallas import tpu as pltpu
from jax.experimental.pallas import tpu_sc as plsc

B = 16384
D = 128
L = 16
NC = 2
NS = 16
NW = NC * NS
ROWS_PER_W = B // NW
CHUNK = 128
NCHUNK = ROWS_PER_W // CHUNK
NBUF = 2


def _rsqrt_newton(x):
    i = lax.bitcast_convert_type(x, jnp.int32)
    i = jnp.int32(0x5F3759DF) - lax.shift_right_logical(i, 1)
    y = lax.bitcast_convert_type(i, jnp.float32)
    half_x = jnp.float32(0.5) * x
    for _ in range(3):
        y = y * (jnp.float32(1.5) - half_x * y * y)
    return y


def _make_sc_kernel():
    mesh = plsc.VectorSubcoreMesh(core_axis_name="c", subcore_axis_name="s")

    @functools.partial(
        pl.kernel,
        mesh=mesh,
        compiler_params=pltpu.CompilerParams(needs_layout_passes=False),
        out_type=jax.ShapeDtypeStruct((B,), jnp.float32),
        scratch_types=[
            pltpu.VMEM((NCHUNK, CHUNK), jnp.int32),
            pltpu.VMEM((NBUF, CHUNK, D), jnp.float32),
            pltpu.VMEM((NBUF, CHUNK, D), jnp.float32),
            pltpu.VMEM((NBUF, CHUNK, D), jnp.float32),
            pltpu.VMEM((NCHUNK, CHUNK), jnp.float32),
            pltpu.SemaphoreType.DMA,
            pltpu.SemaphoreType.DMA,
            pltpu.SemaphoreType.DMA,
            pltpu.SemaphoreType.DMA,
        ],
    )
    def sc_kernel(src_hbm, ids_hbm, dst_hbm, table_hbm, out_hbm,
                  idx_v, src_v, dst_v, rel_v, out_v,
                  idx_sem, buf_sem0, buf_sem1, out_sem):
        wid = lax.axis_index("s") * NC + lax.axis_index("c")
        base = wid * ROWS_PER_W
        buf_sems = (buf_sem0, buf_sem1)

        idx_descs = [
            pltpu.async_copy(
                ids_hbm.at[pl.ds(base + c * CHUNK, CHUNK)], idx_v.at[c], idx_sem
            )
            for c in range(NCHUNK)
        ]

        def start_chunk(c, b):
            r0 = base + c * CHUNK
            idx_descs[c].wait()
            sem = buf_sems[b]
            return [
                pltpu.async_copy(table_hbm.at[idx_v.at[c]], rel_v.at[b], sem),
                pltpu.async_copy(src_hbm.at[pl.ds(r0, CHUNK)], src_v.at[b], sem),
                pltpu.async_copy(dst_hbm.at[pl.ds(r0, CHUNK)], dst_v.at[b], sem),
            ]

        in_descs = [start_chunk(0, 0), start_chunk(1, 1)]
        out_descs = []
        for c in range(NCHUNK):
            bsel = c % NBUF
            for d in in_descs[c]:
                d.wait()
            sv, dv, rv = src_v.at[bsel], dst_v.at[bsel], rel_v.at[bsel]
            for g in range(CHUNK // L):
                rows = jnp.arange(L, dtype=jnp.int32) + jnp.int32(g * L)

                def body(j, acc):
                    cols = jnp.full((L,), j, dtype=jnp.int32)
                    s = plsc.load_gather(sv, [rows, cols])
                    r = plsc.load_gather(rv, [rows, cols])
                    d = plsc.load_gather(dv, [rows, cols])
                    t = s + r - d
                    return acc + t * t

                acc = lax.fori_loop(0, D, body, jnp.zeros((L,), jnp.float32),
                                    unroll=4)
                x = jnp.maximum(acc, jnp.float32(1e-30))
                out_v[c, pl.ds(g * L, L)] = -(x * _rsqrt_newton(x))
            out_descs.append(
                pltpu.async_copy(
                    out_v.at[c], out_hbm.at[pl.ds(base + c * CHUNK, CHUNK)],
                    out_sem,
                )
            )
            if c + NBUF < NCHUNK:
                in_descs.append(start_chunk(c + NBUF, bsel))
        for d in out_descs:
            d.wait()

    return sc_kernel


_SC_KERNEL = _make_sc_kernel()


def kernel(src_emb, rel_ids, dst_emb, rel_table):
    ids = rel_ids.astype(jnp.int32)
    return _SC_KERNEL(src_emb, ids, dst_emb, rel_table)


# speedup vs baseline: 3.6247x; 3.0534x over previous
import functools

import jax
import jax.numpy as jnp
from jax import lax
from jax.experimental import pallas as pl
from jax.experimental.pallas import tpu as pltpu
from jax.experimental.pallas import tpu_sc as plsc

B = 16384
D = 128
L = 16
NC = 2
NS = 16
NW = NC * NS
ROWS_PER_W = B // NW
CHUNK = 128
NCHUNK = ROWS_PER_W // CHUNK
NBUF = 2


def _rsqrt_newton(x):
    i = lax.bitcast_convert_type(x, jnp.int32)
    i = jnp.int32(0x5F3759DF) - lax.shift_right_logical(i, 1)
    y = lax.bitcast_convert_type(i, jnp.float32)
    half_x = jnp.float32(0.5) * x
    for _ in range(3):
        y = y * (jnp.float32(1.5) - half_x * y * y)
    return y


def _make_sc_kernel():
    mesh = plsc.VectorSubcoreMesh(core_axis_name="c", subcore_axis_name="s")

    @functools.partial(
        pl.kernel,
        mesh=mesh,
        compiler_params=pltpu.CompilerParams(needs_layout_passes=False),
        out_type=jax.ShapeDtypeStruct((B,), jnp.float32),
        scratch_types=[
            pltpu.VMEM((NCHUNK, CHUNK), jnp.int32),
            pltpu.VMEM((NBUF, CHUNK, D), jnp.float32),
            pltpu.VMEM((NBUF, CHUNK, D), jnp.float32),
            pltpu.VMEM((NBUF, CHUNK, D), jnp.float32),
            pltpu.VMEM((NCHUNK, CHUNK), jnp.float32),
            pltpu.SemaphoreType.DMA,
            pltpu.SemaphoreType.DMA,
            pltpu.SemaphoreType.DMA,
            pltpu.SemaphoreType.DMA,
        ],
    )
    def sc_kernel(src_hbm, ids_hbm, dst_hbm, table_hbm, out_hbm,
                  idx_v, src_v, dst_v, rel_v, out_v,
                  idx_sem, buf_sem0, buf_sem1, out_sem):
        wid = lax.axis_index("s") * NC + lax.axis_index("c")
        base = wid * ROWS_PER_W
        buf_sems = (buf_sem0, buf_sem1)

        idx_descs = [
            pltpu.async_copy(
                ids_hbm.at[pl.ds(base + c * CHUNK, CHUNK)], idx_v.at[c], idx_sem
            )
            for c in range(NCHUNK)
        ]

        def start_chunk(c, b):
            r0 = base + c * CHUNK
            idx_descs[c].wait()
            sem = buf_sems[b]
            return [
                pltpu.async_copy(table_hbm.at[idx_v.at[c]], rel_v.at[b], sem),
                pltpu.async_copy(src_hbm.at[pl.ds(r0, CHUNK)], src_v.at[b], sem),
                pltpu.async_copy(dst_hbm.at[pl.ds(r0, CHUNK)], dst_v.at[b], sem),
            ]

        in_descs = [start_chunk(0, 0), start_chunk(1, 1)]
        out_descs = []
        for c in range(NCHUNK):
            bsel = c % NBUF
            for d in in_descs[c]:
                d.wait()
            sv, dv, rv = src_v.at[bsel], dst_v.at[bsel], rel_v.at[bsel]
            lane = jnp.arange(L, dtype=jnp.int32)
            for g in range(CHUNK // L):
                rows = lane + jnp.int32(g * L)

                def body(j, acc):
                    cols = (lane + j) & jnp.int32(D - 1)
                    s = plsc.load_gather(sv, [rows, cols])
                    r = plsc.load_gather(rv, [rows, cols])
                    d = plsc.load_gather(dv, [rows, cols])
                    t = s + r - d
                    return acc + t * t

                acc = lax.fori_loop(0, D, body, jnp.zeros((L,), jnp.float32),
                                    unroll=4)
                x = jnp.maximum(acc, jnp.float32(1e-30))
                out_v[c, pl.ds(g * L, L)] = -(x * _rsqrt_newton(x))
            out_descs.append(
                pltpu.async_copy(
                    out_v.at[c], out_hbm.at[pl.ds(base + c * CHUNK, CHUNK)],
                    out_sem,
                )
            )
            if c + NBUF < NCHUNK:
                in_descs.append(start_chunk(c + NBUF, bsel))
        for d in out_descs:
            d.wait()

    return sc_kernel


_SC_KERNEL = _make_sc_kernel()


def kernel(src_emb, rel_ids, dst_emb, rel_table):
    ids = rel_ids.astype(jnp.int32)
    return _SC_KERNEL(src_emb, ids, dst_emb, rel_table)
